# baseline (speedup 1.0000x reference)
import functools

import jax
import jax.numpy as jnp
from jax import lax
from jax.experimental import pallas as pl
from jax.experimental.pallas import tpu as pltpu
from jax.experimental.pallas import tpu_sc as plsc

B = 8
L_BPE = 2048
BPE_PER_WORD = 4
PH_PER_WORD = 2
N_WORDS = L_BPE // BPE_PER_WORD
NB = B * N_WORDS
D = 256
V_PH_PAD = 128

NC = 2
NS = 16
NW = NC * NS
CHUNK = 64

H = 1
NB_H = NB // H
ROWS_H = BPE_PER_WORD * NB_H
NCHUNK = ROWS_H // NW // CHUNK

WB = 1024
GRID = NB_H // WB


def _sc_gather(idx, table):
    mesh = plsc.VectorSubcoreMesh(core_axis_name="c", subcore_axis_name="s")
    nbuf = min(4, NCHUNK)

    @functools.partial(
        pl.kernel,
        mesh=mesh,
        out_type=jax.ShapeDtypeStruct((ROWS_H, D // 2), jnp.int32),
        scratch_types=(
            [pltpu.VMEM((NCHUNK, CHUNK), jnp.int32)]
            + [pltpu.VMEM((CHUNK, D), jnp.int32)] * nbuf
            + [pltpu.VMEM((CHUNK, D // 2), jnp.int32)] * nbuf
            + [pltpu.SemaphoreType.DMA] * (2 * nbuf)
        ),
    )
    def k(idx_hbm, table_hbm, out_hbm, idx_v, *rest):
        bufs = rest[:nbuf]
        obufs = rest[nbuf:2 * nbuf]
        gsems = rest[2 * nbuf:3 * nbuf]
        osems = rest[3 * nbuf:]
        wid = lax.axis_index("s") * NC + lax.axis_index("c")
        base = wid * (NCHUNK * CHUNK)
        pltpu.sync_copy(idx_hbm.at[wid], idx_v)
        gathers = [None] * NCHUNK
        outs = [None] * NCHUNK
        for c in range(min(nbuf - 1, NCHUNK)):
            gathers[c] = pltpu.async_copy(table_hbm.at[idx_v.at[c]], bufs[c % nbuf], gsems[c % nbuf])
        for c in range(NCHUNK):
            gathers[c].wait()
            nxt = c + nbuf - 1
            if nxt < NCHUNK:
                gathers[nxt] = pltpu.async_copy(
                    table_hbm.at[idx_v.at[nxt]], bufs[nxt % nbuf], gsems[nxt % nbuf]
                )
            if c >= nbuf:
                outs[c - nbuf].wait()
            buf, obuf = bufs[c % nbuf], obufs[c % nbuf]

            one = jnp.int32(1)
            half = jnp.int32(0x7FFF)
            s16 = jnp.int32(16)

            def _rnb(u):
                rnd = u + half + (lax.shift_right_logical(u, s16) & one)
                return lax.shift_right_logical(rnd, s16)

            def conv_row(r, carry):
                for g in range(D // 32):
                    ra = _rnb(buf[r, pl.ds(32 * g, 16)])
                    rb = _rnb(buf[r, pl.ds(32 * g + 16, 16)])
                    obuf[r, pl.ds(16 * g, 16)] = lax.shift_left(rb, s16) | ra
                return carry

            lax.fori_loop(0, CHUNK, conv_row, 0)
            outs[c] = pltpu.async_copy(
                obuf, out_hbm.at[pl.ds(base + c * CHUNK, CHUNK)], osems[c % nbuf]
            )
        for c in range(max(0, NCHUNK - nbuf), NCHUNK):
            outs[c].wait()

    return k(idx, table)


def _pack_perm():
    import numpy as np
    j = np.arange(D)
    off = j % 32
    pos = (j // 32) * 32 + np.where(off < 16, 2 * off, 2 * (off - 16) + 1)
    return np.argsort(pos)


_PREC = lax.Precision.DEFAULT


def _dot(a, b):
    return jnp.dot(a, b, preferred_element_type=jnp.float32, precision=_PREC)


_G1 = -2.0 * 0.7978845608028654 * 1.4426950408889634
_G3 = _G1 * 0.044715


def _gelu(x):
    x2 = x * x
    e = jnp.exp2(x * (_G1 + _G3 * x2))
    return x * lax.reciprocal(1.0 + e)


def _tc_body(g_ref, id0_ref, id1_ref, pht_ref, wph_ref, wbpe_ref, wtext_ref,
             out_ref, p2_ref):
    @pl.when(pl.program_id(0) == 0)
    def _():
        p2_ref[...] = _dot(_gelu(_dot(pht_ref[...], wph_ref[...])), wtext_ref[...])

    wbpe = wbpe_ref[...]
    acc = _gelu(_dot(g_ref[0][...].astype(jnp.float32), wbpe))
    for j in range(1, BPE_PER_WORD):
        acc = acc + _gelu(_dot(g_ref[j][...].astype(jnp.float32), wbpe))
    bm2 = _dot(acc * (1.0 / BPE_PER_WORD), wtext_ref[...])

    iota = lax.broadcasted_iota(jnp.int32, (WB, V_PH_PAD), 1)
    p2 = p2_ref[...]
    out = None
    for id_ref in (id0_ref, id1_ref):
        ids = jnp.broadcast_to(id_ref[...], (WB, V_PH_PAD))
        oh = (ids == iota).astype(jnp.float32)
        t = _gelu(_dot(oh, p2) + bm2)
        out = t if out is None else out + t
    out_ref[...] = out * (1.0 / PH_PER_WORD)


def _tc_fused(gathered, ids0, ids1, ph_table_pad, W_ph, W_bpe, W_text):
    full = lambda shape: pl.BlockSpec(shape, lambda i: tuple(0 for _ in shape))
    return pl.pallas_call(
        _tc_body,
        grid=(GRID,),
        in_specs=[
            pl.BlockSpec((BPE_PER_WORD, WB, D), lambda i: (0, i, 0)),
            pl.BlockSpec((WB, 1), lambda i: (i, 0)),
            pl.BlockSpec((WB, 1), lambda i: (i, 0)),
            full((V_PH_PAD, D)),
            full((D, D)),
            full((D, D)),
            full((D, D)),
        ],
        out_specs=pl.BlockSpec((WB, D), lambda i: (i, 0)),
        out_shape=jax.ShapeDtypeStruct((NB_H, D), jnp.float32),
        scratch_shapes=[pltpu.VMEM((V_PH_PAD, D), jnp.float32)],
    )(gathered, ids0, ids1, ph_table_pad, W_ph, W_bpe, W_text)


def kernel(bpe_ids, phoneme_ids, bpe_table, ph_table, W_bpe, W_ph, W_text):
    idx = bpe_ids.reshape(B, N_WORDS, BPE_PER_WORD).transpose(2, 0, 1)
    idx = idx.reshape(BPE_PER_WORD, H, NB_H).transpose(1, 0, 2)
    idx = idx.reshape(H, NW, NCHUNK, CHUNK)

    ph = phoneme_ids.reshape(B, N_WORDS, PH_PER_WORD)
    ids0 = ph[:, :, 0].reshape(NB, 1)
    ids1 = ph[:, :, 1].reshape(NB, 1)

    ph_table_pad = jnp.zeros((V_PH_PAD, D), jnp.float32).at[:ph_table.shape[0]].set(ph_table)

    W_bpe_p = W_bpe[jnp.asarray(_pack_perm())]

    table_i32 = lax.bitcast_convert_type(bpe_table, jnp.int32)
    gathered = [
        lax.bitcast_convert_type(_sc_gather(idx[h], table_i32), jnp.bfloat16)
        .reshape(BPE_PER_WORD, NB_H, D)
        for h in range(H)
    ]
    outs = [
        _tc_fused(
            gathered[h],
            lax.dynamic_slice_in_dim(ids0, h * NB_H, NB_H),
            lax.dynamic_slice_in_dim(ids1, h * NB_H, NB_H),
            ph_table_pad, W_ph, W_bpe_p, W_text,
        )
        for h in range(H)
    ]
    return jnp.concatenate(outs, axis=0) if H > 1 else outs[0]


# speedup vs baseline: 3.6366x; 3.6366x over previous
import functools

import jax
import jax.numpy as jnp
from jax import lax
from jax.experimental import pallas as pl
from jax.experimental.pallas import tpu as pltpu
from jax.experimental.pallas import tpu_sc as plsc

B = 8
L_BPE = 2048
BPE_PER_WORD = 4
PH_PER_WORD = 2
N_WORDS = L_BPE // BPE_PER_WORD
NB = B * N_WORDS
D = 256
V_PH_PAD = 128

NC = 2
NS = 16
NW = NC * NS
CHUNK = 64

H = 1
NB_H = NB // H
ROWS_H = BPE_PER_WORD * NB_H
NCHUNK = ROWS_H // NW // CHUNK

WB = 1024
GRID = NB_H // WB


def _sc_gather(idx, table):
    mesh = plsc.VectorSubcoreMesh(core_axis_name="c", subcore_axis_name="s")
    nbuf = min(6, NCHUNK)

    @functools.partial(
        pl.kernel,
        mesh=mesh,
        out_type=jax.ShapeDtypeStruct((ROWS_H, D), jnp.float32),
        scratch_types=(
            [pltpu.VMEM((NCHUNK, CHUNK), jnp.int32)]
            + [pltpu.VMEM((CHUNK, D), jnp.float32)] * nbuf
            + [pltpu.SemaphoreType.DMA] * (2 * nbuf)
        ),
    )
    def k(idx_hbm, table_hbm, out_hbm, idx_v, *rest):
        bufs = rest[:nbuf]
        gsems = rest[nbuf:2 * nbuf]
        osems = rest[2 * nbuf:]
        wid = lax.axis_index("s") * NC + lax.axis_index("c")
        base = wid * (NCHUNK * CHUNK)
        pltpu.sync_copy(idx_hbm.at[wid], idx_v)
        gathers = [None] * NCHUNK
        outs = [None] * NCHUNK
        waited_out = [False] * NCHUNK
        for c in range(min(nbuf - 1, NCHUNK)):
            gathers[c] = pltpu.async_copy(table_hbm.at[idx_v.at[c]], bufs[c % nbuf], gsems[c % nbuf])
        for c in range(NCHUNK):
            gathers[c].wait()
            outs[c] = pltpu.async_copy(
                bufs[c % nbuf], out_hbm.at[pl.ds(base + c * CHUNK, CHUNK)],
                osems[c % nbuf]
            )
            nxt = c + nbuf - 1
            if nxt < NCHUNK:
                if c >= 1:
                    outs[c - 1].wait()
                    waited_out[c - 1] = True
                gathers[nxt] = pltpu.async_copy(
                    table_hbm.at[idx_v.at[nxt]], bufs[nxt % nbuf], gsems[nxt % nbuf]
                )
        for c in range(NCHUNK):
            if not waited_out[c]:
                outs[c].wait()

    return k(idx, table)


_PREC = lax.Precision.DEFAULT


def _dot(a, b):
    return jnp.dot(a, b, preferred_element_type=jnp.float32, precision=_PREC)


_G1 = -2.0 * 0.7978845608028654 * 1.4426950408889634
_G3 = _G1 * 0.044715


def _gelu(x):
    x2 = x * x
    e = jnp.exp2(x * (_G1 + _G3 * x2))
    return x * lax.reciprocal(1.0 + e)


def _tc_body(g_ref, id0_ref, id1_ref, pht_ref, wph_ref, wbpe_ref, wtext_ref,
             out_ref, p2_ref):
    @pl.when(pl.program_id(0) == 0)
    def _():
        p2_ref[...] = _dot(_gelu(_dot(pht_ref[...], wph_ref[...])), wtext_ref[...])

    wbpe = wbpe_ref[...]
    acc = _gelu(_dot(g_ref[0], wbpe))
    for j in range(1, BPE_PER_WORD):
        acc = acc + _gelu(_dot(g_ref[j], wbpe))
    bm2 = _dot(acc * (1.0 / BPE_PER_WORD), wtext_ref[...])

    iota = lax.broadcasted_iota(jnp.int32, (WB, V_PH_PAD), 1)
    p2 = p2_ref[...]
    out = None
    for id_ref in (id0_ref, id1_ref):
        ids = jnp.broadcast_to(id_ref[...], (WB, V_PH_PAD))
        oh = (ids == iota).astype(jnp.float32)
        t = _gelu(_dot(oh, p2) + bm2)
        out = t if out is None else out + t
    out_ref[...] = out * (1.0 / PH_PER_WORD)


def _tc_fused(gathered, ids0, ids1, ph_table_pad, W_ph, W_bpe, W_text):
    full = lambda shape: pl.BlockSpec(shape, lambda i: tuple(0 for _ in shape))
    return pl.pallas_call(
        _tc_body,
        grid=(GRID,),
        in_specs=[
            pl.BlockSpec((BPE_PER_WORD, WB, D), lambda i: (0, i, 0)),
            pl.BlockSpec((WB, 1), lambda i: (i, 0)),
            pl.BlockSpec((WB, 1), lambda i: (i, 0)),
            full((V_PH_PAD, D)),
            full((D, D)),
            full((D, D)),
            full((D, D)),
        ],
        out_specs=pl.BlockSpec((WB, D), lambda i: (i, 0)),
        out_shape=jax.ShapeDtypeStruct((NB_H, D), jnp.float32),
        scratch_shapes=[pltpu.VMEM((V_PH_PAD, D), jnp.float32)],
    )(gathered, ids0, ids1, ph_table_pad, W_ph, W_bpe, W_text)


def kernel(bpe_ids, phoneme_ids, bpe_table, ph_table, W_bpe, W_ph, W_text):
    idx = bpe_ids.reshape(B, N_WORDS, BPE_PER_WORD).transpose(2, 0, 1)
    idx = idx.reshape(BPE_PER_WORD, H, NB_H).transpose(1, 0, 2)
    idx = idx.reshape(H, NW, NCHUNK, CHUNK)

    ph = phoneme_ids.reshape(B, N_WORDS, PH_PER_WORD)
    ids0 = ph[:, :, 0].reshape(NB, 1)
    ids1 = ph[:, :, 1].reshape(NB, 1)

    ph_table_pad = jnp.zeros((V_PH_PAD, D), jnp.float32).at[:ph_table.shape[0]].set(ph_table)

    gathered = [
        _sc_gather(idx[h], bpe_table).reshape(BPE_PER_WORD, NB_H, D)
        for h in range(H)
    ]
    outs = [
        _tc_fused(
            gathered[h],
            lax.dynamic_slice_in_dim(ids0, h * NB_H, NB_H),
            lax.dynamic_slice_in_dim(ids1, h * NB_H, NB_H),
            ph_table_pad, W_ph, W_bpe, W_text,
        )
        for h in range(H)
    ]
    return jnp.concatenate(outs, axis=0) if H > 1 else outs[0]
